# baseline (speedup 1.0000x reference)
import functools

import jax
import jax.numpy as jnp
from jax import lax
from jax.experimental import pallas as pl
from jax.experimental.pallas import tpu as pltpu
from jax.experimental.pallas import tpu_sc as plsc

_TOPK = 2048
_INT_MIN = -2147483648
_NROWS = 64
_KV = 32768
_NW = 32
_ROWS_PER_W = _NROWS // _NW
_PAD = 16


def _scores_kernel(h_ref, k_ref, wq_ref, wh_ref, out_ref):
    bf = jnp.bfloat16
    f32 = jnp.float32
    S = h_ref.shape[1]
    H = wh_ref.shape[1]
    hd = k_ref.shape[2]

    hb = h_ref[0].astype(bf)
    q32 = jax.lax.dot_general(hb, wq_ref[...].astype(bf),
                              (((1,), (0,)), ((), ())),
                              preferred_element_type=f32)
    qb = q32.astype(bf)
    q2 = jnp.concatenate([qb[s:s + 1, i * hd:(i + 1) * hd]
                          for s in range(S) for i in range(H)],
                         axis=0)
    kb = k_ref[0].astype(bf)
    kq = jax.lax.dot_general(q2, kb, (((1,), (1,)), ((), ())),
                             preferred_element_type=f32)
    s_rel = jnp.maximum(kq, 0.0).astype(bf)

    w32 = jax.lax.dot_general(hb, wh_ref[...].astype(bf),
                              (((1,), (0,)), ((), ())),
                              preferred_element_type=f32)
    wb = w32.astype(bf)
    io_i = jax.lax.broadcasted_iota(jnp.int32, (H, H * S), 0)
    io_j = jax.lax.broadcasted_iota(jnp.int32, (H, H * S), 1)
    rep = (io_j % H == io_i).astype(bf)
    wrep = jax.lax.dot_general(wb, rep, (((1,), (0,)), ((), ())),
                               preferred_element_type=f32)
    io_s = jax.lax.broadcasted_iota(jnp.int32, (S, H * S), 0)
    io_j2 = jax.lax.broadcasted_iota(jnp.int32, (S, H * S), 1)
    w2 = jnp.where(io_j2 // H == io_s, wrep.astype(bf), bf(0))

    out_ref[0] = jax.lax.dot_general(w2, s_rel, (((1,), (0,)), ((), ())),
                                     preferred_element_type=f32)


def _select_kernel(scores_ref, masked_ref, skey_ref, thr_ref):
    bf = jnp.bfloat16
    f32 = jnp.float32
    R = scores_ref.shape[0]
    KV = scores_ref.shape[1]
    imin = jnp.int32(_INT_MIN)
    topk_f = f32(_TOPK)

    s = scores_ref[...]
    bits = jax.lax.bitcast_convert_type(s, jnp.int32)
    skey = jnp.where(bits >= 0, bits, bits ^ jnp.int32(0x7FFFFFFF))
    ones = jnp.ones((KV, 1), bf)

    def cnt(mask_bool):
        return jax.lax.dot_general(mask_bool.astype(bf), ones,
                                   (((1,), (0,)), ((), ())),
                                   preferred_element_type=f32)

    def tbody(t, thr):
        cand = thr | (jnp.int32(1) << (31 - t))
        c = cnt(skey >= (cand ^ imin))
        return jnp.where(c >= topk_f, cand, thr)

    thr_u = jax.lax.fori_loop(0, 32, tbody, jnp.zeros((R, 1), jnp.int32))
    thr_s = thr_u ^ imin
    tn = topk_f - cnt(skey > thr_s)
    tie = skey == thr_s
    idxs = jax.lax.broadcasted_iota(jnp.int32, (R, KV), 1)

    def cbody(t, c):
        cand = c | (jnp.int32(1) << (14 - t))
        n = cnt(tie & (idxs < cand))
        return jnp.where(n <= tn - 1.0, cand, c)

    cstar = jax.lax.fori_loop(0, 15, cbody, jnp.zeros((R, 1), jnp.int32))

    selected = (skey > thr_s) | (tie & (idxs <= cstar))
    masked_ref[...] = jnp.where(selected, s, s - 1e9)
    skey_ref[...] = jnp.where(selected, skey, imin)
    thr_ref[...] = jnp.broadcast_to(thr_s, (R, 16))


def _scores_and_mask(hidden_states, k_cache, wq, w_head):
    B, S, D = hidden_states.shape
    KV, hd = k_cache.shape[1], k_cache.shape[2]
    H = w_head.shape[1]
    C = 4
    scores = pl.pallas_call(
        _scores_kernel,
        grid=(B, C),
        in_specs=[
            pl.BlockSpec((1, S, D), lambda b, c: (b, 0, 0)),
            pl.BlockSpec((1, KV // C, hd), lambda b, c: (b, c, 0)),
            pl.BlockSpec((D, H * hd), lambda b, c: (0, 0)),
            pl.BlockSpec((D, H), lambda b, c: (0, 0)),
        ],
        out_specs=pl.BlockSpec((1, S, KV // C), lambda b, c: (b, 0, c)),
        out_shape=jax.ShapeDtypeStruct((B, S, KV), jnp.float32),
    )(hidden_states, k_cache, wq, w_head)
    scores = scores.reshape(B * S, KV)
    masked, skey_out, thr_rep = pl.pallas_call(
        _select_kernel,
        out_shape=[
            jax.ShapeDtypeStruct((B * S, KV), jnp.float32),
            jax.ShapeDtypeStruct((B * S, KV), jnp.int32),
            jax.ShapeDtypeStruct((B * S, 16), jnp.int32),
        ],
    )(scores)
    return masked, skey_out, thr_rep


def _sc_topk_body(skey_hbm, thr_hbm, out_hbm,
                  row_v, key_a, idx_a, key_b, idx_b, thr_v):
    wid = lax.axis_index("s") * 2 + lax.axis_index("c")
    lane = lax.iota(jnp.int32, 16)
    imin = jnp.int32(_INT_MIN)
    zero_v = jnp.zeros((16,), jnp.int32)

    def popc(mask_bool):
        return plsc.all_reduce_population_count(mask_bool)

    def row_body(t, _):
        r = wid * _ROWS_PER_W + t
        pltpu.sync_copy(skey_hbm.at[r], row_v)
        pltpu.sync_copy(thr_hbm.at[r], thr_v)
        thr = thr_v[...]

        def comp_body(j, carry):
            off, c1 = carry
            v = row_v[pl.ds(j * 16, 16)]
            m = (v >= thr) & (off < _TOPK)
            ukey = v ^ imin
            ps = jnp.cumsum(m.astype(jnp.int32))
            pos = off + ps - 1
            plsc.store_scatter(key_a, [pos], ukey, mask=m)
            plsc.store_scatter(idx_a, [pos], lane + j * 16, mask=m)
            return off + popc(m), c1 + popc(m & ((ukey & 1) == 1))

        _, c1 = lax.fori_loop(0, _KV // 16, comp_body, (zero_v, zero_v),
                              unroll=8)

        def make_dist(src_k, src_i, dst_k, dst_i):
            def dist_body(j, carry):
                o1, o0, cn, p = carry
                k = src_k[pl.ds(j * 16, 16)]
                iv = src_i[pl.ds(j * 16, 16)]
                b = (k >> p) & 1
                ps = jnp.cumsum(b)
                pos = jnp.where(b == 1, o1 + ps - 1, o0 + lane - ps)
                plsc.store_scatter(dst_k, [pos], k)
                plsc.store_scatter(dst_i, [pos], iv)
                n1 = popc(b == 1)
                cn = cn + popc(((k >> (p + 1)) & 1) == 1)
                return o1 + n1, o0 + 16 - n1, cn, p
            return dist_body

        def pair_body(i, c1):
            p = i * 2
            _, _, cn, _ = lax.fori_loop(
                0, _TOPK // 16, make_dist(key_a, idx_a, key_b, idx_b),
                (zero_v, c1, zero_v, p), unroll=8)
            _, _, cn2, _ = lax.fori_loop(
                0, _TOPK // 16, make_dist(key_b, idx_b, key_a, idx_a),
                (zero_v, cn, zero_v, p + 1), unroll=8)
            return cn2

        lax.fori_loop(0, 16, pair_body, c1)
        pltpu.sync_copy(idx_a.at[pl.ds(0, _TOPK)], out_hbm.at[r])
        return 0

    lax.fori_loop(0, _ROWS_PER_W, row_body, 0)


@functools.partial(
    pl.kernel,
    mesh=plsc.VectorSubcoreMesh(core_axis_name="c", subcore_axis_name="s"),
    compiler_params=pltpu.CompilerParams(needs_layout_passes=False),
    out_type=jax.ShapeDtypeStruct((_NROWS, _TOPK), jnp.int32),
    scratch_types=[
        pltpu.VMEM((_KV,), jnp.int32),
        pltpu.VMEM((_TOPK + _PAD,), jnp.int32),
        pltpu.VMEM((_TOPK + _PAD,), jnp.int32),
        pltpu.VMEM((_TOPK + _PAD,), jnp.int32),
        pltpu.VMEM((_TOPK + _PAD,), jnp.int32),
        pltpu.VMEM((16,), jnp.int32),
    ],
)
def _sc_topk(skey_hbm, thr_hbm, out_hbm,
             row_v, key_a, idx_a, key_b, idx_b, thr_v):
    _sc_topk_body(skey_hbm, thr_hbm, out_hbm,
                  row_v, key_a, idx_a, key_b, idx_b, thr_v)


def kernel(hidden_states, k_cache, wq, w_head):
    B, S, _ = hidden_states.shape
    masked, skey_out, thr_rep = _scores_and_mask(
        hidden_states, k_cache, wq, w_head)
    topk_idx = _sc_topk(skey_out, thr_rep)
    return masked.reshape(B, S, _KV), topk_idx.reshape(B, S, _TOPK)
